# baseline (speedup 1.0000x reference)
import functools

import jax
import jax.numpy as jnp
from jax import lax
from jax.experimental import pallas as pl
from jax.experimental.pallas import tpu as pltpu
from jax.experimental.pallas import tpu_sc as plsc

_L = 16


def _build_sc_call(B, S, D):
    mesh = plsc.VectorSubcoreMesh(core_axis_name="c", subcore_axis_name="s",
                                  num_cores=1)
    n_sub = 16
    per_batch = n_sub // B
    chunk = S // per_batch

    @functools.partial(
        pl.kernel,
        mesh=mesh,
        out_type=jax.ShapeDtypeStruct((B, 2, D), jnp.float32),
        scratch_types=[
            pltpu.VMEM((chunk,), jnp.int32),
            pltpu.VMEM((chunk,), jnp.int32),
            pltpu.VMEM((1, D), jnp.float32),
            pltpu.VMEM((1, D), jnp.float32),
            pltpu.SMEM((1,), jnp.int32),
            pltpu.SemaphoreType.DMA,
            pltpu.SemaphoreType.DMA,
            pltpu.SemaphoreType.DMA,
            pltpu.SemaphoreType.DMA,
        ],
    )
    def sc_kernel(x_hbm, am_hbm, tt_hbm, out_hbm,
                  am_v, tt_v, cls_v, sep_v, count_s,
                  sem_am, sem_tt, sem_cls, sem_sep):
        wid = lax.axis_index("s")
        b = wid // per_batch
        q = lax.rem(wid, per_batch)
        leader_wid = b * per_batch
        is_leader = q == 0
        is_cls = q == 1

        cp_am = pltpu.async_copy(am_hbm.at[b, pl.ds(q * chunk, chunk)],
                                 am_v, sem_am)
        cp_tt = pltpu.async_copy(tt_hbm.at[b, pl.ds(q * chunk, chunk)],
                                 tt_v, sem_tt)

        @pl.when(is_cls)
        def _():
            pltpu.async_copy(x_hbm.at[b, pl.ds(0, 1)],
                             out_hbm.at[b, pl.ds(0, 1)], sem_cls)

        @pl.when(is_leader)
        def _():
            count_s[0] = 0
        plsc.subcore_barrier()

        cp_am.wait()
        cp_tt.wait()

        nvec = chunk // _L
        acc = [jnp.zeros((_L,), jnp.int32) for _ in range(4)]
        for i in range(0, nvec, 4):
            for j in range(4):
                a = am_v[pl.ds((i + j) * _L, _L)]
                t = tt_v[pl.ds((i + j) * _L, _L)]
                acc[j] = acc[j] + (a ^ t)
        accv = acc[0] + acc[1] + acc[2] + acc[3]
        partial = accv[0]
        for lane in range(1, _L):
            partial = partial + accv[lane]
        tagged = (partial << 4) + 1
        old = plsc.fetch_and_add(count_s.at[0], tagged, subcore_id=leader_wid)
        is_last = (old & 15) == per_batch - 1

        @pl.when(is_cls)
        def _():
            pltpu.make_async_copy(x_hbm.at[b, pl.ds(0, 1)],
                                  out_hbm.at[b, pl.ds(0, 1)], sem_cls).wait()

        @pl.when(is_last)
        def _():
            total = (old + tagged) >> 4
            sep = total - 1
            sep = jnp.where(sep < 0, sep + S, sep)

            pltpu.sync_copy(x_hbm.at[b, pl.ds(sep, 1)],
                            out_hbm.at[b, pl.ds(1, 1)])

    return sc_kernel


def kernel(x, attention_mask, token_type_ids):
    B, S, D = x.shape
    am = attention_mask.astype(jnp.int32)
    tt = token_type_ids.astype(jnp.int32)
    out3 = _build_sc_call(B, S, D)(x, am, tt)
    return out3.reshape(B, 2 * D)


# speedup vs baseline: 1.0142x; 1.0142x over previous
import functools

import jax
import jax.numpy as jnp
from jax import lax
from jax.experimental import pallas as pl
from jax.experimental.pallas import tpu as pltpu
from jax.experimental.pallas import tpu_sc as plsc

_L = 16


def _build_sc_call(B, S, D):
    mesh = plsc.VectorSubcoreMesh(core_axis_name="c", subcore_axis_name="s",
                                  num_cores=1)
    n_sub = 16
    per_batch = n_sub // B
    chunk = S // per_batch

    @functools.partial(
        pl.kernel,
        mesh=mesh,
        out_type=jax.ShapeDtypeStruct((B, 2, D), jnp.float32),
        scratch_types=[
            pltpu.VMEM((chunk,), jnp.int32),
            pltpu.VMEM((chunk,), jnp.int32),
            pltpu.VMEM((1, D), jnp.float32),
            pltpu.VMEM((1, D), jnp.float32),
            pltpu.SMEM((1,), jnp.int32),
            pltpu.SemaphoreType.DMA,
            pltpu.SemaphoreType.DMA,
            pltpu.SemaphoreType.DMA,
            pltpu.SemaphoreType.DMA,
        ],
    )
    def sc_kernel(x_hbm, am_hbm, tt_hbm, out_hbm,
                  am_v, tt_v, cls_v, sep_v, count_s,
                  sem_am, sem_tt, sem_cls, sem_sep):
        wid = lax.axis_index("s")
        b = wid // per_batch
        q = lax.rem(wid, per_batch)
        leader_wid = b * per_batch
        is_leader = q == 0
        is_cls = q == 1

        cp_am = pltpu.async_copy(am_hbm.at[b, pl.ds(q * chunk, chunk)],
                                 am_v, sem_am)
        cp_tt = pltpu.async_copy(tt_hbm.at[b, pl.ds(q * chunk, chunk)],
                                 tt_v, sem_tt)

        @pl.when(is_cls)
        def _():
            pltpu.async_copy(x_hbm.at[b, pl.ds(0, 1)], cls_v, sem_cls)

        @pl.when(is_leader)
        def _():
            count_s[0] = 0
        plsc.subcore_barrier()

        cp_am.wait()
        cp_tt.wait()

        nvec = chunk // _L
        unroll = 16

        def body(i, accs):
            base = i * (unroll * _L)
            accs = list(accs)
            for j in range(unroll):
                a = am_v[pl.ds(base + j * _L, _L)]
                t = tt_v[pl.ds(base + j * _L, _L)]
                accs[j % 4] = accs[j % 4] + (a ^ t)
            return tuple(accs)

        acc = lax.fori_loop(0, nvec // unroll, body,
                            tuple(jnp.zeros((_L,), jnp.int32)
                                  for _ in range(4)))
        accv = acc[0] + acc[1] + acc[2] + acc[3]
        partial = accv[0]
        for lane in range(1, _L):
            partial = partial + accv[lane]
        tagged = (partial << 4) + 1
        old = plsc.fetch_and_add(count_s.at[0], tagged, subcore_id=leader_wid)
        is_last = (old & 15) == per_batch - 1

        @pl.when(is_cls)
        def _():
            pltpu.make_async_copy(x_hbm.at[b, pl.ds(0, 1)], cls_v,
                                  sem_cls).wait()
            pltpu.sync_copy(cls_v, out_hbm.at[b, pl.ds(0, 1)])

        @pl.when(is_last)
        def _():
            total = (old + tagged) >> 4
            sep = total - 1
            sep = jnp.where(sep < 0, sep + S, sep)

            pltpu.async_copy(x_hbm.at[b, pl.ds(sep, 1)], sep_v,
                             sem_sep).wait()
            pltpu.sync_copy(sep_v, out_hbm.at[b, pl.ds(1, 1)])

    return sc_kernel


def kernel(x, attention_mask, token_type_ids):
    B, S, D = x.shape
    am = attention_mask.astype(jnp.int32)
    tt = token_type_ids.astype(jnp.int32)
    out3 = _build_sc_call(B, S, D)(x, am, tt)
    return out3.reshape(B, 2 * D)


# speedup vs baseline: 1.0894x; 1.0742x over previous
import functools

import jax
import jax.numpy as jnp
from jax import lax
from jax.experimental import pallas as pl
from jax.experimental.pallas import tpu as pltpu
from jax.experimental.pallas import tpu_sc as plsc

_L = 16


def _build_sc_call(B, S, D):
    mesh = plsc.VectorSubcoreMesh(core_axis_name="c", subcore_axis_name="s",
                                  num_cores=1)
    n_sub = 16
    per_batch = n_sub // B
    chunk = S // per_batch

    @functools.partial(
        pl.kernel,
        mesh=mesh,
        out_type=jax.ShapeDtypeStruct((B, 2 * D), jnp.float32),
        scratch_types=[
            pltpu.VMEM((chunk,), jnp.int32),
            pltpu.VMEM((chunk,), jnp.int32),
            pltpu.VMEM((D,), jnp.float32),
            pltpu.VMEM((D,), jnp.float32),
            pltpu.SMEM((1,), jnp.int32),
            pltpu.SemaphoreType.DMA,
            pltpu.SemaphoreType.DMA,
            pltpu.SemaphoreType.DMA,
            pltpu.SemaphoreType.DMA,
        ],
    )
    def sc_kernel(x_hbm, am_hbm, tt_hbm, out_hbm,
                  am_v, tt_v, cls_v, sep_v, count_s,
                  sem_am, sem_tt, sem_cls, sem_sep):
        wid = lax.axis_index("s")
        b = wid // per_batch
        q = lax.rem(wid, per_batch)
        leader_wid = b * per_batch
        is_leader = q == 0
        is_cls = q == 1

        cp_am = pltpu.async_copy(am_hbm.at[b, pl.ds(q * chunk, chunk)],
                                 am_v, sem_am)
        cp_tt = pltpu.async_copy(tt_hbm.at[b, pl.ds(q * chunk, chunk)],
                                 tt_v, sem_tt)

        @pl.when(is_cls)
        def _():
            pltpu.async_copy(x_hbm.at[b, 0], cls_v, sem_cls)

        @pl.when(is_leader)
        def _():
            count_s[0] = 0
        plsc.subcore_barrier()

        cp_am.wait()
        cp_tt.wait()

        nvec = chunk // _L
        acc = [jnp.zeros((_L,), jnp.int32) for _ in range(4)]
        for i in range(0, nvec, 4):
            for j in range(4):
                a = am_v[pl.ds((i + j) * _L, _L)]
                t = tt_v[pl.ds((i + j) * _L, _L)]
                acc[j] = acc[j] + (a ^ t)
        accv = acc[0] + acc[1] + acc[2] + acc[3]
        partial = accv[0]
        for lane in range(1, _L):
            partial = partial + accv[lane]
        tagged = (partial << 4) + 1
        old = plsc.fetch_and_add(count_s.at[0], tagged, subcore_id=leader_wid)
        is_last = (old & 15) == per_batch - 1

        @pl.when(is_cls)
        def _():
            pltpu.make_async_copy(x_hbm.at[b, 0], cls_v, sem_cls).wait()
            pltpu.sync_copy(cls_v, out_hbm.at[b, pl.ds(0, D)])

        @pl.when(is_last)
        def _():
            total = (old + tagged) >> 4
            sep = total - 1
            sep = jnp.where(sep < 0, sep + S, sep)

            pltpu.async_copy(x_hbm.at[b, sep], sep_v, sem_sep).wait()
            pltpu.sync_copy(sep_v, out_hbm.at[b, pl.ds(D, D)])

    return sc_kernel


def kernel(x, attention_mask, token_type_ids):
    B, S, D = x.shape
    am = attention_mask.astype(jnp.int32)
    tt = token_type_ids.astype(jnp.int32)
    return _build_sc_call(B, S, D)(x, am, tt)


# speedup vs baseline: 1.1014x; 1.0111x over previous
import functools

import jax
import jax.numpy as jnp
from jax import lax
from jax.experimental import pallas as pl
from jax.experimental.pallas import tpu as pltpu
from jax.experimental.pallas import tpu_sc as plsc

_L = 16


def _build_sc_call(B, S, D):
    mesh = plsc.VectorSubcoreMesh(core_axis_name="c", subcore_axis_name="s",
                                  num_cores=1)
    n_sub = 16
    per_batch = n_sub // B
    chunk = S // per_batch

    @functools.partial(
        pl.kernel,
        mesh=mesh,
        out_type=jax.ShapeDtypeStruct((B, 2 * D), jnp.float32),
        scratch_types=[
            pltpu.VMEM((chunk,), jnp.int32),
            pltpu.VMEM((chunk,), jnp.int32),
            pltpu.VMEM((D,), jnp.float32),
            pltpu.VMEM((D,), jnp.float32),
            pltpu.SMEM((1,), jnp.int32),
            pltpu.SemaphoreType.DMA,
            pltpu.SemaphoreType.DMA,
            pltpu.SemaphoreType.DMA,
            pltpu.SemaphoreType.DMA,
        ],
    )
    def sc_kernel(x_hbm, am_hbm, tt_hbm, out_hbm,
                  am_v, tt_v, cls_v, sep_v, count_s,
                  sem_am, sem_tt, sem_cls, sem_sep):
        wid = lax.axis_index("s")
        b = wid // per_batch
        q = lax.rem(wid, per_batch)
        leader_wid = b * per_batch
        is_leader = q == 0
        is_cls = q == 1

        cp_am = pltpu.async_copy(am_hbm.at[b, pl.ds(q * chunk, chunk)],
                                 am_v, sem_am)
        cp_tt = pltpu.async_copy(tt_hbm.at[b, pl.ds(q * chunk, chunk)],
                                 tt_v, sem_tt)

        @pl.when(is_cls)
        def _():
            pltpu.async_copy(x_hbm.at[b, 0], cls_v, sem_cls)

        @pl.when(is_leader)
        def _():
            count_s[0] = 0
        plsc.subcore_barrier()

        cp_am.wait()
        cp_tt.wait()

        nvec = chunk // _L
        acc = [jnp.zeros((_L,), jnp.int32) for _ in range(4)]
        for i in range(0, nvec, 4):
            for j in range(4):
                a = am_v[pl.ds((i + j) * _L, _L)]
                t = tt_v[pl.ds((i + j) * _L, _L)]
                acc[j] = acc[j] + (a ^ t)
        accv = acc[0] + acc[1] + acc[2] + acc[3]
        lanes = lax.iota(jnp.int32, _L)
        for shift in (8, 4, 2, 1):
            accv = accv + lax.gather(
                accv, (lanes ^ shift).reshape(_L, 1),
                lax.GatherDimensionNumbers(offset_dims=(),
                                           collapsed_slice_dims=(0,),
                                           start_index_map=(0,)),
                slice_sizes=(1,),
                mode=lax.GatherScatterMode.PROMISE_IN_BOUNDS)
        partial = accv[0]
        tagged = (partial << 4) + 1
        old = plsc.fetch_and_add(count_s.at[0], tagged, subcore_id=leader_wid)
        is_last = (old & 15) == per_batch - 1

        @pl.when(is_cls)
        def _():
            pltpu.make_async_copy(x_hbm.at[b, 0], cls_v, sem_cls).wait()
            pltpu.sync_copy(cls_v, out_hbm.at[b, pl.ds(0, D)])

        @pl.when(is_last)
        def _():
            total = (old + tagged) >> 4
            sep = total - 1
            sep = jnp.where(sep < 0, sep + S, sep)

            pltpu.async_copy(x_hbm.at[b, sep], sep_v, sem_sep).wait()
            pltpu.sync_copy(sep_v, out_hbm.at[b, pl.ds(D, D)])

    return sc_kernel


def kernel(x, attention_mask, token_type_ids):
    B, S, D = x.shape
    am = attention_mask.astype(jnp.int32)
    tt = token_type_ids.astype(jnp.int32)
    return _build_sc_call(B, S, D)(x, am, tt)
